# baseline (speedup 1.0000x reference)
import functools

import jax
import jax.numpy as jnp
from jax import lax
from jax.experimental import pallas as pl
from jax.experimental.pallas import tpu as pltpu
from jax.experimental.pallas import tpu_sc as plsc

TOP_K = 2
TM = 256
DF_BLK = 128

SC_CORES = 2
SC_SUBCORES = 16
NW = SC_CORES * SC_SUBCORES
LANES = 16


def _sc_gather_call(x, row_token_2d, n_rows, hidden):
    rows_per_w = n_rows // NW
    chunk = 16
    n_chunks = rows_per_w // chunk
    idx_rows = rows_per_w // LANES

    mesh = plsc.VectorSubcoreMesh(
        core_axis_name="c", subcore_axis_name="s",
        num_cores=SC_CORES, num_subcores=SC_SUBCORES)

    @functools.partial(
        pl.kernel, mesh=mesh,
        out_type=jax.ShapeDtypeStruct((n_rows, hidden), jnp.float32),
        scratch_types=[
            pltpu.VMEM((idx_rows, LANES), jnp.int32),
            pltpu.VMEM((chunk, hidden), jnp.float32),
            pltpu.VMEM((chunk, hidden), jnp.float32),
            pltpu.SemaphoreType.DMA,
            pltpu.SemaphoreType.DMA,
        ],
    )
    def gk(x_hbm, idx_hbm, out_hbm, idx_v, buf0, buf1, sem0, sem1):
        wid = lax.axis_index("s") * SC_CORES + lax.axis_index("c")
        base_row = wid * rows_per_w
        pltpu.sync_copy(idx_hbm.at[wid], idx_v)
        bufs = (buf0, buf1)
        sems = (sem0, sem1)
        descs = [None, None]
        descs[0] = pltpu.async_copy(x_hbm.at[idx_v.at[0]], buf0, sem0)
        for c in range(n_chunks):
            cur = c % 2
            if c + 1 < n_chunks:
                descs[(c + 1) % 2] = pltpu.async_copy(
                    x_hbm.at[idx_v.at[c + 1]], bufs[(c + 1) % 2],
                    sems[(c + 1) % 2])
            descs[cur].wait()
            pltpu.sync_copy(bufs[cur],
                            out_hbm.at[pl.ds(base_row + c * chunk, chunk)])

    return gk(x, row_token_2d)


def _sc_combine_call(ys, pos0_2d, pos1_2d, n_tok, hidden):
    tok_per_w = n_tok // NW
    chunk = 16
    n_chunks = tok_per_w // chunk
    idx_rows = tok_per_w // LANES

    mesh = plsc.VectorSubcoreMesh(
        core_axis_name="c", subcore_axis_name="s",
        num_cores=SC_CORES, num_subcores=SC_SUBCORES)

    @functools.partial(
        pl.kernel, mesh=mesh,
        out_type=jax.ShapeDtypeStruct((n_tok, hidden), jnp.float32),
        scratch_types=[
            pltpu.VMEM((idx_rows, LANES), jnp.int32),
            pltpu.VMEM((idx_rows, LANES), jnp.int32),
            pltpu.VMEM((chunk, hidden), jnp.float32),
            pltpu.VMEM((chunk, hidden), jnp.float32),
            pltpu.SemaphoreType.DMA,
            pltpu.SemaphoreType.DMA,
        ],
    )
    def ck(ys_hbm, p0_hbm, p1_hbm, out_hbm, p0_v, p1_v, buf0, buf1,
           sem0, sem1):
        wid = lax.axis_index("s") * SC_CORES + lax.axis_index("c")
        base_tok = wid * tok_per_w
        pltpu.sync_copy(p0_hbm.at[wid], p0_v)
        pltpu.sync_copy(p1_hbm.at[wid], p1_v)
        n_vec = hidden // LANES
        for c in range(n_chunks):
            d0 = pltpu.async_copy(ys_hbm.at[p0_v.at[c]], buf0, sem0)
            d1 = pltpu.async_copy(ys_hbm.at[p1_v.at[c]], buf1, sem1)
            d0.wait()
            d1.wait()

            def add_row(r, _):
                def add_vec(v, _):
                    sl = pl.ds(v * LANES, LANES)
                    buf0[r, sl] = buf0[r, sl] + buf1[r, sl]
                    return 0
                lax.fori_loop(0, n_vec, add_vec, 0)
                return 0
            lax.fori_loop(0, chunk, add_row, 0)
            pltpu.sync_copy(buf0,
                            out_hbm.at[pl.ds(base_tok + c * chunk, chunk)])

    return ck(ys, pos0_2d, pos1_2d)


def _ffn_kernel(tile_expert_ref, tile_flag_ref,
                xs_ref, w1g_ref, w1u_ref, w2_ref, w_ref,
                ys_ref, *, n_j):
    j = pl.program_id(1)

    @pl.when(tile_flag_ref[pl.program_id(0)] != 0)
    def _active():
        xs = xs_ref[...]
        gate = lax.dot_general(
            xs, w1g_ref[0, 0], (((1,), (1,)), ((), ())),
            preferred_element_type=jnp.float32)
        up = lax.dot_general(
            xs, w1u_ref[0, 0], (((1,), (1,)), ((), ())),
            preferred_element_type=jnp.float32)
        act = gate * jax.nn.sigmoid(gate) * up
        yj = lax.dot_general(
            act, w2_ref[0], (((1,), (1,)), ((), ())),
            preferred_element_type=jnp.float32)

        @pl.when(j == 0)
        def _init():
            ys_ref[...] = yj

        @pl.when(j > 0)
        def _acc():
            ys_ref[...] += yj

        @pl.when(j == n_j - 1)
        def _weight():
            ys_ref[...] *= w_ref[...]


def kernel(x, router_logits, w1, w2):
    n_tok, hidden = x.shape
    n_exp = w1.shape[0]
    d_ff = w2.shape[2]

    probs = jax.nn.softmax(router_logits.astype(jnp.float32), axis=-1)
    topk_w, topk_idx = lax.top_k(probs, TOP_K)
    topk_w = topk_w / jnp.sum(topk_w, axis=-1, keepdims=True)

    n_asn = n_tok * TOP_K
    e_flat = topk_idx.reshape(-1).astype(jnp.int32)
    w_flat = topk_w.reshape(-1)
    t_flat = jnp.repeat(jnp.arange(n_tok, dtype=jnp.int32), TOP_K)

    order = jnp.argsort(e_flat)
    e_s = e_flat[order]
    t_s = t_flat[order]
    w_s = w_flat[order]

    counts = jnp.bincount(e_flat, length=n_exp)
    padded = ((counts + TM - 1) // TM) * TM
    pstart = jnp.cumsum(padded) - padded
    gstart = jnp.cumsum(counts) - counts
    rank = jnp.arange(n_asn, dtype=jnp.int32) - gstart[e_s].astype(jnp.int32)
    dest = pstart[e_s].astype(jnp.int32) + rank

    n_rows = n_asn + n_exp * TM
    n_tiles = n_rows // TM
    row_token = jnp.zeros((n_rows,), jnp.int32).at[dest].set(t_s)
    row_weight = jnp.zeros((n_rows, 1), jnp.float32).at[dest, 0].set(w_s)

    flat_pos = jnp.zeros((n_asn,), jnp.int32).at[order].set(dest)
    pos = flat_pos.reshape(n_tok, TOP_K)
    pos0 = pos[:, 0].reshape(NW, n_tok // NW // LANES, LANES)
    pos1 = pos[:, 1].reshape(NW, n_tok // NW // LANES, LANES)

    tile_start = jnp.arange(n_tiles, dtype=jnp.int32) * TM
    total_padded = jnp.sum(padded).astype(jnp.int32)
    tile_flag = (tile_start < total_padded).astype(jnp.int32)
    pend = (pstart + padded).astype(jnp.int32)
    tile_expert = jnp.clip(
        jnp.searchsorted(pend, tile_start, side='right'), 0, n_exp - 1
    ).astype(jnp.int32)

    xs = _sc_gather_call(x, row_token.reshape(NW, n_rows // NW // LANES, LANES),
                         n_rows, hidden)

    w1r = w1.reshape(n_exp, 2, d_ff, hidden)
    n_j = d_ff // DF_BLK

    grid_spec = pltpu.PrefetchScalarGridSpec(
        num_scalar_prefetch=2,
        grid=(n_tiles, n_j),
        in_specs=[
            pl.BlockSpec((TM, hidden), lambda i, j, te, tf: (i, 0)),
            pl.BlockSpec((1, 1, DF_BLK, hidden),
                         lambda i, j, te, tf: (te[i], 0, j, 0)),
            pl.BlockSpec((1, 1, DF_BLK, hidden),
                         lambda i, j, te, tf: (te[i], 1, j, 0)),
            pl.BlockSpec((1, hidden, DF_BLK),
                         lambda i, j, te, tf: (te[i], 0, j)),
            pl.BlockSpec((TM, 1), lambda i, j, te, tf: (i, 0)),
        ],
        out_specs=pl.BlockSpec((TM, hidden), lambda i, j, te, tf: (i, 0)),
    )

    ys = pl.pallas_call(
        functools.partial(_ffn_kernel, n_j=n_j),
        grid_spec=grid_spec,
        out_shape=jax.ShapeDtypeStruct((n_rows, hidden), jnp.float32),
        compiler_params=pltpu.CompilerParams(
            dimension_semantics=("arbitrary", "arbitrary")),
    )(tile_expert, tile_flag, xs, w1r, w1r, w2, row_weight)

    out = _sc_combine_call(ys, pos0, pos1, n_tok, hidden)
    return out
